# Initial kernel scaffold; baseline (speedup 1.0000x reference)
import jax
import jax.numpy as jnp
from jax.experimental import pallas as pl


def kernel(inputs, token_table, pos_table):
    raise NotImplementedError("write your pallas kernel here")


# speedup vs baseline: 2.6061x; 2.6061x over previous
import functools

import jax
import jax.numpy as jnp
from jax import lax
from jax.experimental import pallas as pl
from jax.experimental.pallas import tpu as pltpu
from jax.experimental.pallas import tpu_sc as plsc

SEQ = 200
EMB = 64
BATCH = 1024
NC = 2
NS = 16
NW = NC * NS
SEQ_PER_W = BATCH // NW
HALF = SEQ // 2
LANES = 16
SCALE = 8.0


def _sc_embed(idx, token_table, pos_table):
    mesh = plsc.VectorSubcoreMesh(
        core_axis_name="c", subcore_axis_name="s", num_cores=NC, num_subcores=NS
    )

    @functools.partial(
        pl.kernel,
        mesh=mesh,
        compiler_params=pltpu.CompilerParams(use_tc_tiling_on_sc=False),
        out_type=jax.ShapeDtypeStruct((BATCH, SEQ, EMB), jnp.float32),
        scratch_types=[
            pltpu.VMEM((2, HALF), jnp.int32),
            pltpu.VMEM((SEQ, EMB), jnp.float32),
            pltpu.VMEM((SEQ, EMB), jnp.float32),
            pltpu.SemaphoreType.DMA,
        ],
    )
    def k(idx_hbm, tok_hbm, pos_hbm, out_hbm, idx_v, row_v, pos_v, sem):
        wid = lax.axis_index("s") * NC + lax.axis_index("c")
        pltpu.sync_copy(pos_hbm, pos_v)

        def seq_body(i, carry):
            b = wid * SEQ_PER_W + i
            pltpu.sync_copy(idx_hbm.at[b], idx_v)
            cp0 = pltpu.async_copy(
                tok_hbm.at[idx_v.at[0]], row_v.at[pl.ds(0, HALF)], sem
            )
            cp1 = pltpu.async_copy(
                tok_hbm.at[idx_v.at[1]], row_v.at[pl.ds(HALF, HALF)], sem
            )
            cp0.wait()
            cp1.wait()

            def row_body(r, carry2):
                for rr in range(4):
                    row = r * 4 + rr
                    for j in range(EMB // LANES):
                        sl = pl.ds(j * LANES, LANES)
                        row_v[row, sl] = row_v[row, sl] * SCALE + pos_v[row, sl]
                return carry2

            lax.fori_loop(0, SEQ // 4, row_body, 0)
            pltpu.sync_copy(row_v, out_hbm.at[b])
            return carry

        lax.fori_loop(0, SEQ_PER_W, seq_body, 0)

    return k(idx, token_table, pos_table)


def kernel(inputs, token_table, pos_table):
    idx = inputs.astype(jnp.int32).reshape(BATCH, 2, HALF)
    return _sc_embed(
        idx, token_table.astype(jnp.float32), pos_table.astype(jnp.float32)
    )


# speedup vs baseline: 3.1333x; 1.2023x over previous
import functools

import jax
import jax.numpy as jnp
from jax import lax
from jax.experimental import pallas as pl
from jax.experimental.pallas import tpu as pltpu
from jax.experimental.pallas import tpu_sc as plsc

SEQ = 200
EMB = 64
BATCH = 1024
NC = 2
NS = 16
NW = NC * NS
SEQ_PER_W = BATCH // NW
HALF = SEQ // 2
LANES = 16
SCALE = 8.0
NBUF = 4


def _sc_embed(idx, token_table, pos_table):
    mesh = plsc.VectorSubcoreMesh(
        core_axis_name="c", subcore_axis_name="s", num_cores=NC, num_subcores=NS
    )

    @functools.partial(
        pl.kernel,
        mesh=mesh,
        compiler_params=pltpu.CompilerParams(use_tc_tiling_on_sc=False),
        out_type=jax.ShapeDtypeStruct((BATCH, SEQ, EMB), jnp.float32),
        scratch_types=[
            pltpu.VMEM((SEQ_PER_W, 2, HALF), jnp.int32),
            pltpu.VMEM((SEQ, EMB), jnp.float32),
        ]
        + [pltpu.VMEM((SEQ, EMB), jnp.float32) for _ in range(NBUF)]
        + [pltpu.SemaphoreType.DMA for _ in range(2 * NBUF)],
    )
    def k(idx_hbm, tok_hbm, pos_hbm, out_hbm, idx_v, pos_v, *rest):
        bufs = rest[:NBUF]
        gsem = rest[NBUF : 2 * NBUF]
        wsem = rest[2 * NBUF :]
        wid = lax.axis_index("s") * NC + lax.axis_index("c")
        base = wid * SEQ_PER_W
        pltpu.sync_copy(pos_hbm, pos_v)
        pltpu.sync_copy(idx_hbm.at[pl.ds(base, SEQ_PER_W)], idx_v)

        def start_gather(i, b):
            pltpu.async_copy(
                tok_hbm.at[idx_v.at[i, 0]], bufs[b].at[pl.ds(0, HALF)], gsem[b]
            )
            pltpu.async_copy(
                tok_hbm.at[idx_v.at[i, 1]], bufs[b].at[pl.ds(HALF, HALF)], gsem[b]
            )

        def wait_gather(b):
            pltpu.make_async_copy(out_hbm.at[0], bufs[b], gsem[b]).wait()

        def wait_wb(b):
            pltpu.make_async_copy(bufs[b], out_hbm.at[0], wsem[b]).wait()

        def compute(b):
            buf = bufs[b]

            def row_body(r, carry):
                for rr in range(4):
                    row = r * 4 + rr
                    for j in range(EMB // LANES):
                        sl = pl.ds(j * LANES, LANES)
                        buf[row, sl] = buf[row, sl] * SCALE + pos_v[row, sl]
                return carry

            lax.fori_loop(0, SEQ // 4, row_body, 0)

        for i in range(NBUF - 1):
            start_gather(i, i)

        def outer(o, carry):
            for phase in range(NBUF):
                i = NBUF * o + phase
                b = phase
                nb = (phase + NBUF - 1) % NBUF
                if phase == 0:
                    @pl.when(o >= 1)
                    def _():
                        wait_wb(nb)

                    start_gather(i + NBUF - 1, nb)
                else:
                    @pl.when(o <= SEQ_PER_W // NBUF - 2)
                    def _():
                        wait_wb(nb)
                        start_gather(i + NBUF - 1, nb)

                wait_gather(b)
                compute(b)
                pltpu.async_copy(bufs[b], out_hbm.at[base + i], wsem[b])
            return carry

        lax.fori_loop(0, SEQ_PER_W // NBUF, outer, 0)
        for b in range(NBUF):
            wait_wb(b)

    return k(idx, token_table, pos_table)


def kernel(inputs, token_table, pos_table):
    idx = inputs.astype(jnp.int32).reshape(BATCH, 2, HALF)
    return _sc_embed(
        idx, token_table.astype(jnp.float32), pos_table.astype(jnp.float32)
    )
